# baseline (speedup 1.0000x reference)
import jax
import jax.numpy as jnp
from jax.experimental import pallas as pl
from jax.experimental.pallas import tpu as pltpu

TOPK = 64
TQ = 256
SQ = 2048
SK = 2048
D = 1024
NB = 4
NQT = SQ // TQ
INT_MIN = -2147483648


def _sortable(x):
    i = jax.lax.bitcast_convert_type(x, jnp.int32)
    return jnp.where(i < 0, i ^ jnp.int32(0x7FFFFFFF), i)


def _body(q_ref, k_ref, v_ref, out_ref, mstore_ref, colsq_ref):
    b = pl.program_id(0)
    qt = pl.program_id(1)
    slot = jax.lax.rem(b, 2)
    prev = jax.lax.rem(b + 1, 2)

    @pl.when(b < NB)
    def _phase_a():
        s = jax.lax.dot_general(
            q_ref[0],
            k_ref[0],
            dimension_numbers=(((1,), (1,)), ((), ())),
            preferred_element_type=jnp.float32,
        )
        ss = _sortable(s)
        cnt_pos = jnp.sum((ss >= 0).astype(jnp.int32), axis=1, keepdims=True)
        t = jnp.where(cnt_pos >= TOPK, jnp.int32(0), jnp.int32(INT_MIN))
        for bit in range(30, -1, -1):
            trial = t | jnp.int32(1 << bit)
            cnt = jnp.sum((ss >= trial).astype(jnp.int32), axis=1, keepdims=True)
            t = jnp.where(cnt >= TOPK, trial, t)
        masked = jnp.where(ss >= t, s, 0.0)
        mstore_ref[slot, qt] = masked.astype(jnp.bfloat16)

        @pl.when(qt == 0)
        def _init():
            colsq_ref[slot] = jnp.zeros_like(colsq_ref[slot])

        colsq_ref[slot] += jnp.broadcast_to(
            jnp.sum(masked * masked, axis=0)[None, :], (8, SK)
        )

    @pl.when(b >= 1)
    def _phase_b():
        colsq = colsq_ref[prev][0]
        scale = jax.lax.rsqrt(jnp.maximum(colsq, 1e-24))
        att = (mstore_ref[prev, qt].astype(jnp.float32) * scale[None, :]).astype(
            jnp.bfloat16
        )
        out_ref[0] = jax.lax.dot_general(
            att,
            v_ref[0],
            dimension_numbers=(((1,), (0,)), ((), ())),
            preferred_element_type=jnp.float32,
        )


@jax.jit
def kernel(query, key, value):
    qb = query.astype(jnp.bfloat16)
    kb = key.astype(jnp.bfloat16)
    vb = value.astype(jnp.bfloat16)

    out = pl.pallas_call(
        _body,
        grid=(NB + 1, NQT),
        in_specs=[
            pl.BlockSpec((1, TQ, D), lambda b, q: (jnp.minimum(b, NB - 1), q, 0)),
            pl.BlockSpec((1, SK, D), lambda b, q: (jnp.minimum(b, NB - 1), 0, 0)),
            pl.BlockSpec((1, SK, D), lambda b, q: (jnp.maximum(b - 1, 0), 0, 0)),
        ],
        out_specs=pl.BlockSpec((1, TQ, D), lambda b, q: (jnp.maximum(b - 1, 0), q, 0)),
        out_shape=jax.ShapeDtypeStruct((NB, SQ, D), jnp.float32),
        scratch_shapes=[
            pltpu.VMEM((2, NQT, TQ, SK), jnp.bfloat16),
            pltpu.VMEM((2, 8, SK), jnp.float32),
        ],
        compiler_params=pltpu.CompilerParams(
            dimension_semantics=("arbitrary", "arbitrary"),
        ),
    )(qb, kb, vb)
    return out


# speedup vs baseline: 1.2501x; 1.2501x over previous
import jax
import jax.numpy as jnp
from jax.experimental import pallas as pl
from jax.experimental.pallas import tpu as pltpu

TOPK = 64
TQ = 256
SQ = 2048
SK = 2048
D = 1024


def _sortable(x):
    i = jax.lax.bitcast_convert_type(x, jnp.int32)
    return jnp.where(i < 0, i ^ jnp.int32(0x7FFFFFFF), i)


def _phase1_body(q_ref, k_ref, masked_ref, colsq_ref):
    qt = pl.program_id(1)
    s = jax.lax.dot_general(
        q_ref[0],
        k_ref[0],
        dimension_numbers=(((1,), (1,)), ((), ())),
        preferred_element_type=jnp.float32,
    )
    ss = _sortable(s)
    hi = jax.lax.shift_right_arithmetic(ss, 16).astype(jnp.int16)
    lo = ((ss & jnp.int32(0xFFFF)) ^ jnp.int32(0x8000)).astype(jnp.int16)

    def cntm(mask16):
        x = jnp.where(mask16, jnp.bfloat16(1), jnp.bfloat16(0))
        x = x[:, : SK // 2] + x[:, SK // 2 :]
        x = x[:, : SK // 4] + x[:, SK // 4 :]
        x = x[:, : SK // 8] + x[:, SK // 8 :]
        return jnp.sum(
            x.astype(jnp.float32), axis=1, keepdims=True, dtype=jnp.float32
        )

    kf = jnp.float32(TOPK)
    th = jnp.where(cntm(hi >= jnp.int16(0)) >= kf, jnp.int32(0), jnp.int32(-32768))
    for bit in range(14, -1, -1):
        trial = th | jnp.int32(1 << bit)
        th = jnp.where(cntm(hi >= trial.astype(jnp.int16)) >= kf, trial, th)
    th16 = th.astype(jnp.int16)
    cnt_gt = cntm(hi > th16)
    lo_m = jnp.where(hi == th16, lo, jnp.int16(-32768))
    tl = jnp.where(
        cnt_gt + cntm(lo_m >= jnp.int16(0)) >= kf, jnp.int32(0), jnp.int32(-32768)
    )
    for bit in range(14, -1, -1):
        trial = tl | jnp.int32(1 << bit)
        cnt = cnt_gt + cntm(lo_m >= trial.astype(jnp.int16))
        tl = jnp.where(cnt >= kf, trial, tl)
    t = (th << 16) | ((tl & jnp.int32(0xFFFF)) ^ jnp.int32(0x8000))
    masked = jnp.where(ss >= t, s, 0.0)
    masked_ref[0] = masked.astype(jnp.bfloat16)

    @pl.when(qt == 0)
    def _init():
        colsq_ref[...] = jnp.zeros_like(colsq_ref)

    colsq_ref[...] += jnp.broadcast_to(
        jnp.sum(masked * masked, axis=0)[None, None, :], colsq_ref.shape
    )


def _phase2_body(masked_ref, colsq_ref, v_ref, out_ref):
    colsq = colsq_ref[0, 0]
    scale = jax.lax.rsqrt(jnp.maximum(colsq, 1e-24)).astype(jnp.bfloat16)
    att = masked_ref[0] * scale[None, :]
    out_ref[0] = jax.lax.dot_general(
        att,
        v_ref[0],
        dimension_numbers=(((1,), (0,)), ((), ())),
        preferred_element_type=jnp.float32,
    )


@jax.jit
def kernel(query, key, value):
    B = query.shape[0]
    grid = (B, SQ // TQ)
    qb = query.astype(jnp.bfloat16)
    kb = key.astype(jnp.bfloat16)
    vb = value.astype(jnp.bfloat16)

    masked, colsq = pl.pallas_call(
        _phase1_body,
        grid=grid,
        in_specs=[
            pl.BlockSpec((1, TQ, D), lambda b, q: (b, q, 0)),
            pl.BlockSpec((1, SK, D), lambda b, q: (b, 0, 0)),
        ],
        out_specs=[
            pl.BlockSpec((1, TQ, SK), lambda b, q: (b, q, 0)),
            pl.BlockSpec((1, 8, SK), lambda b, q: (b, 0, 0)),
        ],
        out_shape=[
            jax.ShapeDtypeStruct((B, SQ, SK), jnp.bfloat16),
            jax.ShapeDtypeStruct((B, 8, SK), jnp.float32),
        ],
        compiler_params=pltpu.CompilerParams(
            dimension_semantics=("arbitrary", "arbitrary"),
        ),
    )(qb, kb)

    out = pl.pallas_call(
        _phase2_body,
        grid=grid,
        in_specs=[
            pl.BlockSpec((1, TQ, SK), lambda b, q: (b, q, 0)),
            pl.BlockSpec((1, 8, SK), lambda b, q: (b, 0, 0)),
            pl.BlockSpec((1, SK, D), lambda b, q: (b, 0, 0)),
        ],
        out_specs=pl.BlockSpec((1, TQ, D), lambda b, q: (b, q, 0)),
        out_shape=jax.ShapeDtypeStruct((B, SQ, D), jnp.float32),
        compiler_params=pltpu.CompilerParams(
            dimension_semantics=("arbitrary", "arbitrary"),
        ),
    )(masked, colsq, vb)
    return out


# speedup vs baseline: 1.3604x; 1.0882x over previous
import jax
import jax.numpy as jnp
from jax.experimental import pallas as pl
from jax.experimental.pallas import tpu as pltpu

TOPK = 64
TQ = 512
SQ = 2048
SK = 2048
D = 1024


def _sortable(x):
    i = jax.lax.bitcast_convert_type(x, jnp.int32)
    return jnp.where(i < 0, i ^ jnp.int32(0x7FFFFFFF), i)


def _phase1_body(q_ref, k_ref, masked_ref, colsq_ref):
    qt = pl.program_id(1)
    s = jax.lax.dot_general(
        q_ref[0],
        k_ref[0],
        dimension_numbers=(((1,), (1,)), ((), ())),
        preferred_element_type=jnp.float32,
    )
    ss = _sortable(s)
    hi = jax.lax.shift_right_arithmetic(ss, 16).astype(jnp.int16)
    lo = ((ss & jnp.int32(0xFFFF)) ^ jnp.int32(0x8000)).astype(jnp.int16)

    def cntm(mask16):
        x = jnp.where(mask16, jnp.bfloat16(1), jnp.bfloat16(0))
        x = x[:, : SK // 2] + x[:, SK // 2 :]
        x = x[:, : SK // 4] + x[:, SK // 4 :]
        x = x[:, : SK // 8] + x[:, SK // 8 :]
        x = x[:, : SK // 16] + x[:, SK // 16 :]
        return jnp.sum(
            x.astype(jnp.float32), axis=1, keepdims=True, dtype=jnp.float32
        )

    kf = jnp.float32(TOPK)
    th = jnp.where(cntm(hi >= jnp.int16(0)) >= kf, jnp.int32(0), jnp.int32(-32768))
    for bit in range(14, -1, -1):
        trial = th | jnp.int32(1 << bit)
        th = jnp.where(cntm(hi >= trial.astype(jnp.int16)) >= kf, trial, th)
    th16 = th.astype(jnp.int16)
    cnt_gt = cntm(hi > th16)
    lo_m = jnp.where(hi == th16, lo, jnp.int16(-32768))
    tl = jnp.where(
        cnt_gt + cntm(lo_m >= jnp.int16(0)) >= kf, jnp.int32(0), jnp.int32(-32768)
    )
    for bit in range(14, -1, -1):
        trial = tl | jnp.int32(1 << bit)
        cnt = cnt_gt + cntm(lo_m >= trial.astype(jnp.int16))
        tl = jnp.where(cnt >= kf, trial, tl)
    t = (th << 16) | ((tl & jnp.int32(0xFFFF)) ^ jnp.int32(0x8000))
    masked = jnp.where(ss >= t, s, 0.0)
    masked_ref[0] = masked.astype(jnp.bfloat16)

    @pl.when(qt == 0)
    def _init():
        colsq_ref[...] = jnp.zeros_like(colsq_ref)

    colsq_ref[...] += jnp.broadcast_to(
        jnp.sum(masked * masked, axis=0)[None, None, :], colsq_ref.shape
    )


def _phase2_body(masked_ref, colsq_ref, v_ref, out_ref):
    colsq = colsq_ref[0, 0]
    scale = jax.lax.rsqrt(jnp.maximum(colsq, 1e-24)).astype(jnp.bfloat16)
    att = masked_ref[0] * scale[None, :]
    out_ref[0] = jax.lax.dot_general(
        att,
        v_ref[0],
        dimension_numbers=(((1,), (0,)), ((), ())),
        preferred_element_type=jnp.float32,
    )


@jax.jit
def kernel(query, key, value):
    B = query.shape[0]
    grid = (B, SQ // TQ)
    qb = query.astype(jnp.bfloat16)
    kb = key.astype(jnp.bfloat16)
    vb = value.astype(jnp.bfloat16)

    masked, colsq = pl.pallas_call(
        _phase1_body,
        grid=grid,
        in_specs=[
            pl.BlockSpec((1, TQ, D), lambda b, q: (b, q, 0)),
            pl.BlockSpec((1, SK, D), lambda b, q: (b, 0, 0)),
        ],
        out_specs=[
            pl.BlockSpec((1, TQ, SK), lambda b, q: (b, q, 0)),
            pl.BlockSpec((1, 8, SK), lambda b, q: (b, 0, 0)),
        ],
        out_shape=[
            jax.ShapeDtypeStruct((B, SQ, SK), jnp.bfloat16),
            jax.ShapeDtypeStruct((B, 8, SK), jnp.float32),
        ],
        compiler_params=pltpu.CompilerParams(
            dimension_semantics=("arbitrary", "arbitrary"),
        ),
    )(qb, kb)

    out = pl.pallas_call(
        _phase2_body,
        grid=grid,
        in_specs=[
            pl.BlockSpec((1, TQ, SK), lambda b, q: (b, q, 0)),
            pl.BlockSpec((1, 8, SK), lambda b, q: (b, 0, 0)),
            pl.BlockSpec((1, SK, D), lambda b, q: (b, 0, 0)),
        ],
        out_specs=pl.BlockSpec((1, TQ, D), lambda b, q: (b, q, 0)),
        out_shape=jax.ShapeDtypeStruct((B, SQ, D), jnp.float32),
        compiler_params=pltpu.CompilerParams(
            dimension_semantics=("arbitrary", "arbitrary"),
        ),
    )(masked, colsq, vb)
    return out


# speedup vs baseline: 1.5900x; 1.1688x over previous
import jax
import jax.numpy as jnp
from jax.experimental import pallas as pl
from jax.experimental.pallas import tpu as pltpu

TOPK = 64
TQ = 512
SQ = 2048
SK = 2048
D = 1024


def _sortable(x):
    i = jax.lax.bitcast_convert_type(x, jnp.int32)
    return jnp.where(i < 0, i ^ jnp.int32(0x7FFFFFFF), i)


def _phase1_body(q_ref, k_ref, masked_ref, colsq_ref):
    qt = pl.program_id(1)
    s = jax.lax.dot_general(
        q_ref[0],
        k_ref[0],
        dimension_numbers=(((1,), (1,)), ((), ())),
        preferred_element_type=jnp.float32,
    )
    ss = _sortable(s)
    hi = jax.lax.shift_right_arithmetic(ss, 16).astype(jnp.int16)
    lo = ((ss & jnp.int32(0xFFFF)) ^ jnp.int32(0x8000)).astype(jnp.int16)

    def cntm(mask16):
        x = jnp.where(mask16, jnp.bfloat16(1), jnp.bfloat16(0))
        x = x[:, : SK // 2] + x[:, SK // 2 :]
        x = x[:, : SK // 4] + x[:, SK // 4 :]
        x = x[:, : SK // 8] + x[:, SK // 8 :]
        x = x[:, : SK // 16] + x[:, SK // 16 :]
        return jnp.sum(
            x.astype(jnp.float32), axis=1, keepdims=True, dtype=jnp.float32
        )

    kf = jnp.float32(TOPK)
    th = jnp.where(cntm(hi >= jnp.int16(0)) >= kf, jnp.int32(0), jnp.int32(-32768))
    for bit in range(14, -1, -1):
        trial = th | jnp.int32(1 << bit)
        th = jnp.where(cntm(hi >= trial.astype(jnp.int16)) >= kf, trial, th)
    th16 = th.astype(jnp.int16)
    cnt_gt = cntm(hi > th16)
    lo_m = jnp.where(hi == th16, lo, jnp.int16(-32768))
    tl = jnp.where(
        cnt_gt + cntm(lo_m >= jnp.int16(0)) >= kf, jnp.int32(0), jnp.int32(-32768)
    )
    for bit in range(14, -1, -1):
        trial = tl | jnp.int32(1 << bit)
        cnt = cnt_gt + cntm(lo_m >= trial.astype(jnp.int16))
        tl = jnp.where(cnt >= kf, trial, tl)
    t = (th << 16) | ((tl & jnp.int32(0xFFFF)) ^ jnp.int32(0x8000))
    masked = jnp.where(ss >= t, s, 0.0)
    masked_ref[0] = masked.astype(jnp.bfloat16)

    @pl.when(qt == 0)
    def _init():
        colsq_ref[...] = jnp.zeros_like(colsq_ref)

    colsq_ref[...] += jnp.broadcast_to(
        jnp.sum(masked * masked, axis=0)[None, None, :], colsq_ref.shape
    )


def _phase2_body(masked_ref, colsq_ref, v_ref, out_ref):
    colsq = colsq_ref[0, 0]
    scale = jax.lax.rsqrt(jnp.maximum(colsq, 1e-24)).astype(jnp.bfloat16)
    att = masked_ref[0] * scale[None, :]
    out_ref[0] = jax.lax.dot_general(
        att,
        v_ref[0].astype(jnp.bfloat16),
        dimension_numbers=(((1,), (0,)), ((), ())),
        preferred_element_type=jnp.float32,
    )


@jax.jit
def kernel(query, key, value):
    B = query.shape[0]
    grid = (B, SQ // TQ)

    masked, colsq = pl.pallas_call(
        _phase1_body,
        grid=grid,
        in_specs=[
            pl.BlockSpec((1, TQ, D), lambda b, q: (b, q, 0)),
            pl.BlockSpec((1, SK, D), lambda b, q: (b, 0, 0)),
        ],
        out_specs=[
            pl.BlockSpec((1, TQ, SK), lambda b, q: (b, q, 0)),
            pl.BlockSpec((1, 8, SK), lambda b, q: (b, 0, 0)),
        ],
        out_shape=[
            jax.ShapeDtypeStruct((B, SQ, SK), jnp.bfloat16),
            jax.ShapeDtypeStruct((B, 8, SK), jnp.float32),
        ],
        compiler_params=pltpu.CompilerParams(
            dimension_semantics=("arbitrary", "arbitrary"),
        ),
    )(query, key)

    out = pl.pallas_call(
        _phase2_body,
        grid=grid,
        in_specs=[
            pl.BlockSpec((1, TQ, SK), lambda b, q: (b, q, 0)),
            pl.BlockSpec((1, 8, SK), lambda b, q: (b, 0, 0)),
            pl.BlockSpec((1, SK, D), lambda b, q: (b, 0, 0)),
        ],
        out_specs=pl.BlockSpec((1, TQ, D), lambda b, q: (b, q, 0)),
        out_shape=jax.ShapeDtypeStruct((B, SQ, D), jnp.float32),
        compiler_params=pltpu.CompilerParams(
            dimension_semantics=("arbitrary", "arbitrary"),
        ),
    )(masked, colsq, value)
    return out
